# baseline (speedup 1.0000x reference)
import functools

import jax
import jax.numpy as jnp
from jax import lax
from jax.experimental import pallas as pl
from jax.experimental.pallas import tpu as pltpu
from jax.experimental.pallas import tpu_sc as plsc

_N = 10000
_H = 128
_T = 4
_E = 80000
_L_STEPS = (3, 3)
_NC = 2
_NS = 16
_CB = 128
_KCH = 40
_EPS = _KCH * _CB
_EPAD = _EPS * _NS
_TRASH = _N
_RPS = 640
_NACC = _NS * _RPS
_ZR = 128
_BR = 1000
_NL = len(_L_STEPS)


def _sc_mesh():
    return plsc.VectorSubcoreMesh(core_axis_name="c", subcore_axis_name="s")


def _sc_scatter_types(table, src_idx, dst_idx, zrows):
    @functools.partial(
        pl.kernel,
        out_type=jax.ShapeDtypeStruct((_T, _NACC, _H), jnp.float32),
        mesh=_sc_mesh(),
        scratch_types=[
            pltpu.VMEM((_KCH, _CB), jnp.int32),
            pltpu.VMEM((_KCH, _CB), jnp.int32),
            pltpu.VMEM((_CB, _H), jnp.float32),
            pltpu.VMEM((_ZR, _H), jnp.float32),
            pltpu.VMEM_SHARED((_NACC, _H), jnp.float32),
            pltpu.SemaphoreType.DMA,
        ],
    )
    def k(table_h, src_h, dst_h, z_h, out_h, src_v, dst_v, rows_v, zb_v, acc,
          sem):
        c = lax.axis_index("c")
        s = lax.axis_index("s")
        pltpu.sync_copy(z_h, zb_v)
        for p in range(_T // _NC):
            jt = p * _NC + c
            for q in range(_RPS // _ZR):
                pltpu.sync_copy(zb_v, acc.at[pl.ds(s * _RPS + q * _ZR, _ZR)])
            plsc.subcore_barrier()
            pltpu.sync_copy(src_h.at[jt, s], src_v)
            pltpu.sync_copy(dst_h.at[jt, s], dst_v)

            def chunk(i, _):
                pltpu.async_copy(table_h.at[src_v.at[i]], rows_v, sem).wait()
                pltpu.sync_copy(rows_v, acc.at[dst_v.at[i]], add=True)
                return ()

            lax.fori_loop(0, _KCH, chunk, ())
            plsc.subcore_barrier()
            pltpu.sync_copy(acc.at[pl.ds(s * _RPS, _RPS)],
                            out_h.at[jt].at[pl.ds(s * _RPS, _RPS)])

    return k(table, src_idx, dst_idx, zrows)


def _sc_counts(dst_idx, ones_rows, zrows):
    @functools.partial(
        pl.kernel,
        out_type=jax.ShapeDtypeStruct((_T, _NACC, _H), jnp.float32),
        mesh=_sc_mesh(),
        scratch_types=[
            pltpu.VMEM((_KCH, _CB), jnp.int32),
            pltpu.VMEM((_CB, _H), jnp.float32),
            pltpu.VMEM((_ZR, _H), jnp.float32),
            pltpu.VMEM_SHARED((_NACC, _H), jnp.float32),
        ],
    )
    def k(dst_h, one_h, z_h, out_h, dst_v, ones_v, zb_v, acc):
        c = lax.axis_index("c")
        s = lax.axis_index("s")
        pltpu.sync_copy(z_h, zb_v)
        pltpu.sync_copy(one_h, ones_v)
        for p in range(_T // _NC):
            jt = p * _NC + c
            for q in range(_RPS // _ZR):
                pltpu.sync_copy(zb_v, acc.at[pl.ds(s * _RPS + q * _ZR, _ZR)])
            plsc.subcore_barrier()
            pltpu.sync_copy(dst_h.at[jt, s], dst_v)

            def chunk(i, _):
                pltpu.sync_copy(ones_v, acc.at[dst_v.at[i]], add=True)
                return ()

            lax.fori_loop(0, _KCH, chunk, ())
            plsc.subcore_barrier()
            pltpu.sync_copy(acc.at[pl.ds(s * _RPS, _RPS)],
                            out_h.at[jt].at[pl.ds(s * _RPS, _RPS)])

    return k(dst_idx, ones_rows, zrows)


def _bias_body(cnt_ref, bm_ref, o_ref):
    for l in range(_NL):
        acc = jnp.zeros((_BR, _H), jnp.float32)
        for j in range(_T):
            acc = acc + cnt_ref[j][:, 0:1] * bm_ref[l, j][None, :]
        o_ref[l] = acc


def _bias_tables(cnt, b_msg):
    return pl.pallas_call(
        _bias_body,
        grid=(_N // _BR,),
        in_specs=[
            pl.BlockSpec((_T, _BR, _H), lambda i: (0, i, 0)),
            pl.BlockSpec((_NL, _T, _H), lambda i: (0, 0, 0)),
        ],
        out_specs=pl.BlockSpec((_NL, _BR, _H), lambda i: (0, i, 0)),
        out_shape=jax.ShapeDtypeStruct((_NL, _N, _H), jnp.float32),
    )(cnt, b_msg)


def _dense_body(a_ref, t_ref, bc_ref, wm_ref, wih_ref, whh_ref,
                bih_ref, bhh_ref, o_ref):
    h = t_ref[...]
    inc = bc_ref[...]
    for j in range(_T):
        inc = inc + lax.dot(a_ref[j], wm_ref[j], preferred_element_type=jnp.float32)
    gi = lax.dot(inc, wih_ref[...], preferred_element_type=jnp.float32) + bih_ref[...]
    gh = lax.dot(h, whh_ref[...], preferred_element_type=jnp.float32) + bhh_ref[...]
    r = jax.nn.sigmoid(gi[:, :_H] + gh[:, :_H])
    z = jax.nn.sigmoid(gi[:, _H:2 * _H] + gh[:, _H:2 * _H])
    n = jnp.tanh(gi[:, 2 * _H:] + r * gh[:, 2 * _H:])
    o_ref[...] = (1.0 - z) * n + z * h


def _dense_step(A, table, bias_cnt, Wm, WihT, WhhT, bih, bhh):
    return pl.pallas_call(
        _dense_body,
        grid=(_N // _BR,),
        in_specs=[
            pl.BlockSpec((_T, _BR, _H), lambda i: (0, i, 0)),
            pl.BlockSpec((_BR, _H), lambda i: (i, 0)),
            pl.BlockSpec((_BR, _H), lambda i: (i, 0)),
            pl.BlockSpec((_T, _H, _H), lambda i: (0, 0, 0)),
            pl.BlockSpec((_H, 3 * _H), lambda i: (0, 0)),
            pl.BlockSpec((_H, 3 * _H), lambda i: (0, 0)),
            pl.BlockSpec((1, 3 * _H), lambda i: (0, 0)),
            pl.BlockSpec((1, 3 * _H), lambda i: (0, 0)),
        ],
        out_specs=pl.BlockSpec((_BR, _H), lambda i: (i, 0)),
        out_shape=jax.ShapeDtypeStruct((_N, _H), jnp.float32),
    )(A, table, bias_cnt, Wm, WihT, WhhT, bih, bhh)


def kernel(initial_node_representation, adjacency_lists, W_msg, b_msg,
           W_ih, W_hh, b_ih, b_hh):
    table = initial_node_representation

    src = adjacency_lists[:, :, 0]
    dst = adjacency_lists[:, :, 1]
    npad = _EPAD - _E
    src_p = jnp.concatenate(
        [src, jnp.zeros((_T, npad), jnp.int32)], axis=1).reshape(_T, _NS, _KCH, _CB)
    dst_p = jnp.concatenate(
        [dst, jnp.full((_T, npad), _TRASH, jnp.int32)], axis=1).reshape(_T, _NS, _KCH, _CB)
    zrows = jnp.zeros((_ZR, _H), jnp.float32)

    Wm = jnp.swapaxes(W_msg, -1, -2)
    WihT = jnp.swapaxes(W_ih, -1, -2)
    WhhT = jnp.swapaxes(W_hh, -1, -2)
    bih = b_ih[:, None, :]
    bhh = b_hh[:, None, :]

    ones_rows = jnp.ones((_CB, _H), jnp.float32)
    cnt = _sc_counts(dst_p, ones_rows, zrows)
    bias = _bias_tables(cnt, b_msg)

    for l, steps in enumerate(_L_STEPS):
        for _ in range(steps):
            A = _sc_scatter_types(table, src_p, dst_p, zrows)
            table = _dense_step(A, table, bias[l], Wm[l], WihT[l], WhhT[l],
                                bih[l], bhh[l])
    return table


# speedup vs baseline: 1.0360x; 1.0360x over previous
import functools

import jax
import jax.numpy as jnp
from jax import lax
from jax.experimental import pallas as pl
from jax.experimental.pallas import tpu as pltpu
from jax.experimental.pallas import tpu_sc as plsc

_N = 10000
_H = 128
_T = 4
_E = 80000
_L_STEPS = (3, 3)
_NC = 2
_NS = 16
_CB = 128
_KCH = 40
_EPS = _KCH * _CB
_EPAD = _EPS * _NS
_TRASH = _N
_RPS = 640
_NACC = _NS * _RPS
_ZR = 128
_BR = 1000
_NL = len(_L_STEPS)


def _sc_mesh():
    return plsc.VectorSubcoreMesh(core_axis_name="c", subcore_axis_name="s")


def _sc_scatter_types(table, src_idx, dst_idx, zrows):
    @functools.partial(
        pl.kernel,
        out_type=jax.ShapeDtypeStruct((_T, _NACC, _H), jnp.float32),
        mesh=_sc_mesh(),
        scratch_types=[
            pltpu.VMEM((_KCH, _CB), jnp.int32),
            pltpu.VMEM((_KCH, _CB), jnp.int32),
            pltpu.VMEM((_CB, _H), jnp.float32),
            pltpu.VMEM((_CB, _H), jnp.float32),
            pltpu.VMEM_SHARED((_NACC, _H), jnp.float32),
            pltpu.SemaphoreType.DMA,
            pltpu.SemaphoreType.DMA,
        ],
    )
    def k(table_h, src_h, dst_h, z_h, out_h, src_v, dst_v, rows_a, rows_b,
          acc, sem_a, sem_b):
        c = lax.axis_index("c")
        s = lax.axis_index("s")
        for p in range(_T // _NC):
            jt = p * _NC + c
            pltpu.sync_copy(z_h, rows_a)
            for q in range(_RPS // _ZR):
                pltpu.sync_copy(rows_a, acc.at[pl.ds(s * _RPS + q * _ZR, _ZR)])
            plsc.subcore_barrier()
            pltpu.sync_copy(src_h.at[jt, s], src_v)
            pltpu.sync_copy(dst_h.at[jt, s], dst_v)

            def chunk2(kk, _):
                i0 = kk * 2
                ga = pltpu.async_copy(table_h.at[src_v.at[i0]], rows_a, sem_a)
                gb = pltpu.async_copy(table_h.at[src_v.at[i0 + 1]], rows_b,
                                      sem_b)
                ga.wait()
                pltpu.sync_copy(rows_a, acc.at[dst_v.at[i0]], add=True)
                gb.wait()
                pltpu.sync_copy(rows_b, acc.at[dst_v.at[i0 + 1]], add=True)
                return ()

            lax.fori_loop(0, _KCH // 2, chunk2, ())
            plsc.subcore_barrier()
            pltpu.sync_copy(acc.at[pl.ds(s * _RPS, _RPS)],
                            out_h.at[jt].at[pl.ds(s * _RPS, _RPS)])

    return k(table, src_idx, dst_idx, zrows)


def _sc_counts(dst_idx, ones_rows, zrows):
    @functools.partial(
        pl.kernel,
        out_type=jax.ShapeDtypeStruct((_T, _NACC, _H), jnp.float32),
        mesh=_sc_mesh(),
        scratch_types=[
            pltpu.VMEM((_KCH, _CB), jnp.int32),
            pltpu.VMEM((_CB, _H), jnp.float32),
            pltpu.VMEM((_ZR, _H), jnp.float32),
            pltpu.VMEM_SHARED((_NACC, _H), jnp.float32),
        ],
    )
    def k(dst_h, one_h, z_h, out_h, dst_v, ones_v, zb_v, acc):
        c = lax.axis_index("c")
        s = lax.axis_index("s")
        pltpu.sync_copy(z_h, zb_v)
        pltpu.sync_copy(one_h, ones_v)
        for p in range(_T // _NC):
            jt = p * _NC + c
            for q in range(_RPS // _ZR):
                pltpu.sync_copy(zb_v, acc.at[pl.ds(s * _RPS + q * _ZR, _ZR)])
            plsc.subcore_barrier()
            pltpu.sync_copy(dst_h.at[jt, s], dst_v)

            def chunk(i, _):
                pltpu.sync_copy(ones_v, acc.at[dst_v.at[i]], add=True)
                return ()

            lax.fori_loop(0, _KCH, chunk, ())
            plsc.subcore_barrier()
            pltpu.sync_copy(acc.at[pl.ds(s * _RPS, _RPS)],
                            out_h.at[jt].at[pl.ds(s * _RPS, _RPS)])

    return k(dst_idx, ones_rows, zrows)


def _bias_body(cnt_ref, bm_ref, o_ref):
    for l in range(_NL):
        acc = jnp.zeros((_BR, _H), jnp.float32)
        for j in range(_T):
            acc = acc + cnt_ref[j][:, 0:1] * bm_ref[l, j][None, :]
        o_ref[l] = acc


def _bias_tables(cnt, b_msg):
    return pl.pallas_call(
        _bias_body,
        grid=(_N // _BR,),
        in_specs=[
            pl.BlockSpec((_T, _BR, _H), lambda i: (0, i, 0)),
            pl.BlockSpec((_NL, _T, _H), lambda i: (0, 0, 0)),
        ],
        out_specs=pl.BlockSpec((_NL, _BR, _H), lambda i: (0, i, 0)),
        out_shape=jax.ShapeDtypeStruct((_NL, _N, _H), jnp.float32),
    )(cnt, b_msg)


def _dense_body(a_ref, t_ref, bc_ref, wm_ref, wih_ref, whh_ref,
                bih_ref, bhh_ref, o_ref):
    h = t_ref[...]
    inc = bc_ref[...]
    for j in range(_T):
        inc = inc + lax.dot(a_ref[j], wm_ref[j], preferred_element_type=jnp.float32)
    gi = lax.dot(inc, wih_ref[...], preferred_element_type=jnp.float32) + bih_ref[...]
    gh = lax.dot(h, whh_ref[...], preferred_element_type=jnp.float32) + bhh_ref[...]
    r = jax.nn.sigmoid(gi[:, :_H] + gh[:, :_H])
    z = jax.nn.sigmoid(gi[:, _H:2 * _H] + gh[:, _H:2 * _H])
    n = jnp.tanh(gi[:, 2 * _H:] + r * gh[:, 2 * _H:])
    o_ref[...] = (1.0 - z) * n + z * h


def _dense_step(A, table, bias_cnt, Wm, WihT, WhhT, bih, bhh):
    return pl.pallas_call(
        _dense_body,
        grid=(_N // _BR,),
        in_specs=[
            pl.BlockSpec((_T, _BR, _H), lambda i: (0, i, 0)),
            pl.BlockSpec((_BR, _H), lambda i: (i, 0)),
            pl.BlockSpec((_BR, _H), lambda i: (i, 0)),
            pl.BlockSpec((_T, _H, _H), lambda i: (0, 0, 0)),
            pl.BlockSpec((_H, 3 * _H), lambda i: (0, 0)),
            pl.BlockSpec((_H, 3 * _H), lambda i: (0, 0)),
            pl.BlockSpec((1, 3 * _H), lambda i: (0, 0)),
            pl.BlockSpec((1, 3 * _H), lambda i: (0, 0)),
        ],
        out_specs=pl.BlockSpec((_BR, _H), lambda i: (i, 0)),
        out_shape=jax.ShapeDtypeStruct((_N, _H), jnp.float32),
    )(A, table, bias_cnt, Wm, WihT, WhhT, bih, bhh)


def kernel(initial_node_representation, adjacency_lists, W_msg, b_msg,
           W_ih, W_hh, b_ih, b_hh):
    table = initial_node_representation

    src = adjacency_lists[:, :, 0]
    dst = adjacency_lists[:, :, 1]
    npad = _EPAD - _E
    src_p = jnp.concatenate(
        [src, jnp.zeros((_T, npad), jnp.int32)], axis=1).reshape(_T, _NS, _KCH, _CB)
    dst_p = jnp.concatenate(
        [dst, jnp.full((_T, npad), _TRASH, jnp.int32)], axis=1).reshape(_T, _NS, _KCH, _CB)
    zrows = jnp.zeros((_ZR, _H), jnp.float32)

    Wm = jnp.swapaxes(W_msg, -1, -2)
    WihT = jnp.swapaxes(W_ih, -1, -2)
    WhhT = jnp.swapaxes(W_hh, -1, -2)
    bih = b_ih[:, None, :]
    bhh = b_hh[:, None, :]

    ones_rows = jnp.ones((_CB, _H), jnp.float32)
    cnt = _sc_counts(dst_p, ones_rows, zrows)
    bias = _bias_tables(cnt, b_msg)

    for l, steps in enumerate(_L_STEPS):
        for _ in range(steps):
            A = _sc_scatter_types(table, src_p, dst_p, zrows)
            table = _dense_step(A, table, bias[l], Wm[l], WihT[l], WhhT[l],
                                bih[l], bhh[l])
    return table
